# baseline (speedup 1.0000x reference)
import jax
import jax.numpy as jnp
from jax import lax
from jax.experimental import pallas as pl
from jax.experimental.pallas import tpu as pltpu

B = 8
N = 2048
TILE = 512
TPB = N // TILE
NT = B * TPB
M = B * N
EPS = 1e-5
C1, C2, C3 = 256, 512, 1024
BF = jnp.bfloat16
F32 = jnp.float32


def _accum_stats(yb, sm, sq, first):
    ones = jnp.ones((TILE, 1), BF)
    mv = jnp.dot(yb, ones, preferred_element_type=F32)
    sqb = yb * yb
    mq = jnp.dot(sqb, ones, preferred_element_type=F32)

    @pl.when(first)
    def _():
        sm[...] = mv
        sq[...] = mq

    @pl.when(jnp.logical_not(first))
    def _():
        sm[...] += mv
        sq[...] += mq


def _finalize(sm, sq, g, be, scb, shb):
    mean = sm[...] * (1.0 / M)
    var = jnp.maximum(sq[...] * (1.0 / M) - mean * mean, 0.0)
    sc = g * lax.rsqrt(var + EPS)
    sh = be - mean * sc
    zeros = jnp.zeros(scb.shape, F32)
    scb[...] = zeros + sc
    shb[...] = zeros + sh


def _body(xyz_ref, pts_ref, w0a_ref, w0b_ref, w1_ref, w2_ref,
          g0_ref, be0_ref, g1_ref, be1_ref, g2_ref, be2_ref,
          out_ref,
          y0s, y1s, s0m, s0q, s1m, s1q, s2m, s2q,
          sc0b, sh0b, sc1b, sh1b,
          amax, amin, ymax, ymin):
    i = pl.program_id(0)
    t = i % NT
    b = t // TPB
    tt = t % TPB

    @pl.when(i < NT)
    def _phase0():
        xv = xyz_ref[t]
        pv = pts_ref[0]
        u = jnp.dot(w0b_ref[...], pv, preferred_element_type=F32)
        u = u + jnp.dot(w0a_ref[...], xv, preferred_element_type=F32)
        yb = u.astype(BF)
        y0s[t] = yb
        _accum_stats(yb, s0m, s0q, t == 0)

        @pl.when(t == NT - 1)
        def _():
            _finalize(s0m, s0q, g0_ref[...], be0_ref[...], sc0b, sh0b)

    @pl.when(jnp.logical_and(i >= NT, i < 2 * NT))
    def _phase1():
        y0 = y0s[t].astype(F32)
        z = jnp.maximum(y0 * sc0b[...] + sh0b[...], 0.0).astype(BF)
        u = jnp.dot(w1_ref[...], z, preferred_element_type=F32)
        yb = u.astype(BF)
        y1s[t] = yb
        _accum_stats(yb, s1m, s1q, t == 0)

        @pl.when(t == NT - 1)
        def _():
            _finalize(s1m, s1q, g1_ref[...], be1_ref[...], sc1b, sh1b)

    @pl.when(i >= 2 * NT)
    def _phase2():
        y1 = y1s[t].astype(F32)
        z = jnp.maximum(y1 * sc1b[...] + sh1b[...], 0.0).astype(BF)
        u = jnp.dot(w2_ref[...], z, preferred_element_type=F32)
        yb = u.astype(BF)
        _accum_stats(yb, s2m, s2q, t == 0)

        @pl.when(tt == 0)
        def _():
            amax[...] = yb
            amin[...] = yb

        @pl.when(tt != 0)
        def _():
            amax[...] = jnp.maximum(amax[...], yb)
            amin[...] = jnp.minimum(amin[...], yb)

        @pl.when(tt == TPB - 1)
        def _():
            mx = jnp.max(amax[...], axis=1, keepdims=True).astype(F32)
            mn = jnp.min(amin[...], axis=1, keepdims=True).astype(F32)
            lanes = lax.broadcasted_iota(jnp.int32, (C3, B), 1)
            ymax[...] = jnp.where(lanes == b, mx, ymax[...])
            ymin[...] = jnp.where(lanes == b, mn, ymin[...])

        @pl.when(t == NT - 1)
        def _():
            mean = s2m[...] * (1.0 / M)
            var = jnp.maximum(s2q[...] * (1.0 / M) - mean * mean, 0.0)
            sc = g2_ref[...] * lax.rsqrt(var + EPS)
            sh = be2_ref[...] - mean * sc
            ext = jnp.where(sc >= 0.0, ymax[...], ymin[...])
            out_ref[...] = jnp.maximum(ext * sc + sh, 0.0)


def kernel(xyz, points, W0, b0, g0, beta0, W1, b1, g1, beta1, W2, b2, g2, beta2):
    del b0, b1, b2
    xyz_t = xyz.transpose(1, 0, 2).reshape(3, NT, TILE).transpose(1, 0, 2).astype(BF)
    pts = points.astype(BF)
    w0a = W0[:, :3].astype(BF)
    w0b = W0[:, 3:].astype(BF)
    w1 = W1.astype(BF)
    w2 = W2.astype(BF)

    def col(v):
        return v.reshape(-1, 1).astype(F32)

    grid = 3 * NT
    full = lambda shape: pl.BlockSpec(shape, lambda i: tuple(0 for _ in shape))
    out = pl.pallas_call(
        _body,
        grid=(grid,),
        in_specs=[
            full((NT, 3, TILE)),
            pl.BlockSpec((1, C1, TILE),
                         lambda i: (jnp.minimum(i, NT - 1) // TPB, 0,
                                    jnp.minimum(i, NT - 1) % TPB)),
            full((C1, 3)),
            full((C1, C1)),
            full((C2, C1)),
            full((C3, C2)),
            full((C1, 1)),
            full((C1, 1)),
            full((C2, 1)),
            full((C2, 1)),
            full((C3, 1)),
            full((C3, 1)),
        ],
        out_specs=pl.BlockSpec((C3, B), lambda i: (0, 0)),
        out_shape=jax.ShapeDtypeStruct((C3, B), F32),
        scratch_shapes=[
            pltpu.VMEM((NT, C1, TILE), BF),
            pltpu.VMEM((NT, C2, TILE), BF),
            pltpu.VMEM((C1, 1), F32),
            pltpu.VMEM((C1, 1), F32),
            pltpu.VMEM((C2, 1), F32),
            pltpu.VMEM((C2, 1), F32),
            pltpu.VMEM((C3, 1), F32),
            pltpu.VMEM((C3, 1), F32),
            pltpu.VMEM((C1, TILE), F32),
            pltpu.VMEM((C1, TILE), F32),
            pltpu.VMEM((C2, TILE), F32),
            pltpu.VMEM((C2, TILE), F32),
            pltpu.VMEM((C3, TILE), BF),
            pltpu.VMEM((C3, TILE), BF),
            pltpu.VMEM((C3, B), F32),
            pltpu.VMEM((C3, B), F32),
        ],
    )(xyz_t, pts, w0a, w0b, w1, w2,
      col(g0), col(beta0), col(g1), col(beta1), col(g2), col(beta2))

    new_points = out.T.reshape(B, C3, 1)
    new_xyz = jnp.zeros((B, 3, 1), F32)
    return new_xyz, new_points
